# baseline (speedup 1.0000x reference)
import functools

import jax
import jax.numpy as jnp
from jax import lax
from jax.experimental import pallas as pl
from jax.experimental.pallas import tpu as pltpu
from jax.experimental.pallas import tpu_sc as plsc

N = 1_000_000
LANES = 16
NW = 32

NBLK_TC = 4
M = 786_432
BLK_TC = M // NBLK_TC
S = N - M
CHUNK = 64 * (S // (NW * 64))
NB_SC = 4
B_SC = CHUNK // NB_SC
TAIL = S - NW * CHUNK
TAIL_BASE = N - TAIL


def _hc_vec(a, u):
    g = jnp.exp(-a) * (1.0 - u) / u
    i = lax.bitcast_convert_type(g, jnp.int32)
    i = jnp.int32(0x5F3759DF) - lax.shift_right_logical(i, 1)
    y = lax.bitcast_convert_type(i, jnp.float32)
    y = y * (1.5 - 0.5 * g * y * y)
    y = y * (1.5 - 0.5 * g * y * y)
    z = g * g * y
    s = (1.1 - 0.1 * z) / (1.0 + z)
    return jnp.clip(s, 0.0, 1.0)


def _tc_body(a_ref, u_ref, o_ref):
    a = a_ref[...]
    u = u_ref[...]
    g = jnp.exp(-a) * (1.0 - u) / u
    z = g * g * lax.rsqrt(g)
    s = (1.1 - 0.1 * z) / (1.0 + z)
    o_ref[...] = jnp.clip(s, 0.0, 1.0)


_tc_kernel = pl.pallas_call(
    _tc_body,
    grid=(NBLK_TC,),
    in_specs=[
        pl.BlockSpec((BLK_TC,), lambda i: (i,)),
        pl.BlockSpec((BLK_TC,), lambda i: (i,)),
    ],
    out_specs=pl.BlockSpec((BLK_TC,), lambda i: (i,)),
    out_shape=jax.ShapeDtypeStruct((N,), jnp.float32),
)


_mesh = plsc.VectorSubcoreMesh(core_axis_name="c", subcore_axis_name="s")


@functools.partial(
    pl.kernel,
    mesh=_mesh,
    out_type=jax.ShapeDtypeStruct((S,), jnp.float32),
    scratch_types=[
        pltpu.VMEM((B_SC,), jnp.float32),
        pltpu.VMEM((B_SC,), jnp.float32),
        pltpu.VMEM((B_SC,), jnp.float32),
        pltpu.VMEM((B_SC,), jnp.float32),
        pltpu.VMEM((B_SC,), jnp.float32),
        pltpu.VMEM((B_SC,), jnp.float32),
        pltpu.VMEM((TAIL,), jnp.float32),
        pltpu.VMEM((TAIL,), jnp.float32),
        pltpu.VMEM((TAIL,), jnp.float32),
        pltpu.SemaphoreType.DMA,
        pltpu.SemaphoreType.DMA,
        pltpu.SemaphoreType.DMA,
        pltpu.SemaphoreType.DMA,
    ],
)
def _sc_kernel(a_hbm, u_hbm, o_hbm,
               a_v0, a_v1, u_v0, u_v1, o_v0, o_v1,
               at_v, ut_v, ot_v,
               sin0, sin1, sout0, sout1):
    wid = lax.axis_index("s") * 2 + lax.axis_index("c")
    base = M + wid * CHUNK
    obase = wid * CHUNK
    a_v = (a_v0, a_v1)
    u_v = (u_v0, u_v1)
    o_v = (o_v0, o_v1)
    sin = (sin0, sin1)
    sout = (sout0, sout1)

    def in_copies(k):
        s = k & 1
        ca = pltpu.make_async_copy(a_hbm.at[pl.ds(base + k * B_SC, B_SC)],
                                   a_v[s], sin[s])
        cu = pltpu.make_async_copy(u_hbm.at[pl.ds(base + k * B_SC, B_SC)],
                                   u_v[s], sin[s])
        return ca, cu

    def out_copy(k):
        s = k & 1
        return pltpu.make_async_copy(o_v[s],
                                     o_hbm.at[pl.ds(obase + k * B_SC, B_SC)],
                                     sout[s])

    for c in in_copies(0):
        c.start()
    for c in in_copies(1):
        c.start()

    for k in range(NB_SC):
        s = k & 1
        for c in in_copies(k):
            c.wait()
        if k >= 2:
            out_copy(k - 2).wait()

        @plsc.parallel_loop(0, B_SC, step=LANES, unroll=8)
        def _compute(i):
            sl = pl.ds(i, LANES)
            o_v[s][sl] = _hc_vec(a_v[s][sl], u_v[s][sl])

        out_copy(k).start()
        if k + 2 < NB_SC:
            for c in in_copies(k + 2):
                c.start()

    out_copy(NB_SC - 2).wait()
    out_copy(NB_SC - 1).wait()

    @pl.when(wid == 0)
    def _tail():
        pltpu.sync_copy(a_hbm.at[pl.ds(TAIL_BASE, TAIL)], at_v)
        pltpu.sync_copy(u_hbm.at[pl.ds(TAIL_BASE, TAIL)], ut_v)

        @plsc.parallel_loop(0, TAIL, step=LANES, unroll=4)
        def _tcomp(j):
            sl = pl.ds(j, LANES)
            ot_v[sl] = _hc_vec(at_v[sl], ut_v[sl])

        pltpu.sync_copy(ot_v, o_hbm.at[pl.ds(S - TAIL, TAIL)])


def kernel(log_alpha, u, current_iter):
    sc_out = _sc_kernel(log_alpha, u)
    tc_out = _tc_kernel(log_alpha, u)
    return lax.dynamic_update_slice(tc_out, sc_out, (M,))
